# baseline (speedup 1.0000x reference)
import functools

import jax
import jax.numpy as jnp
from jax import lax
from jax.experimental import pallas as pl
from jax.experimental.pallas import tpu as pltpu
from jax.experimental.pallas import tpu_sc as plsc

T = 8192
D = 1024
E = 8
F = 4096

TB = 1024
NBR = T // TB

TM = 64
NSLOT = T + E * TM
NB = NSLOT // TM
KBLK = 512
KC = F // KBLK

NW = 32
TPW = T // NW
CH = 32
NCH = TPW // CH


def _router_body(x_ref, wr_ref, br_ref,
                 probs_ref, sel_ref, rank_ref, offs_ref, bex_ref,
                 carry_ref):
    b = pl.program_id(0)
    x = x_ref[...]
    logits = jnp.dot(x, wr_ref[...], preferred_element_type=jnp.float32)
    logits = logits + br_ref[...]
    m = jnp.max(logits, axis=1, keepdims=True)
    ex = jnp.exp(logits - m)
    probs = ex / jnp.sum(ex, axis=1, keepdims=True)
    probs_ref[...] = probs
    sel = jnp.argmax(probs, axis=1).astype(jnp.int32)
    sel_ref[0, 0, :] = sel

    onehot = (jax.lax.broadcasted_iota(jnp.int32, (TB, E), 1)
              == sel[:, None]).astype(jnp.float32)
    tri = (jax.lax.broadcasted_iota(jnp.int32, (TB, TB), 1)
           <= jax.lax.broadcasted_iota(jnp.int32, (TB, TB), 0)
           ).astype(jnp.float32)
    incl = jnp.dot(tri, onehot, preferred_element_type=jnp.float32)
    rank_in = jnp.sum(incl * onehot, axis=1) - 1.0

    @pl.when(b == 0)
    def _():
        carry_ref[...] = jnp.zeros_like(carry_ref)

    carry = carry_ref[...]
    prev = jnp.sum(onehot * carry, axis=1)
    rank_ref[0, 0, :] = (rank_in + prev).astype(jnp.int32)
    carry_ref[...] = carry + jnp.sum(onehot, axis=0, keepdims=True)

    @pl.when(b == NBR - 1)
    def _():
        cnt = carry_ref[...].astype(jnp.int32)
        pc = ((cnt + (TM - 1)) // TM) * TM
        upper = (jax.lax.broadcasted_iota(jnp.int32, (E, E), 0)
                 < jax.lax.broadcasted_iota(jnp.int32, (E, E), 1)
                 ).astype(jnp.float32)
        offs = jnp.dot(pc.astype(jnp.float32), upper,
                       preferred_element_type=jnp.float32)
        offs_ref[...] = offs.astype(jnp.int32)
        endf = offs + pc.astype(jnp.float32)
        eye = (jax.lax.broadcasted_iota(jnp.int32, (E, E), 0)
               == jax.lax.broadcasted_iota(jnp.int32, (E, E), 1)
               ).astype(jnp.float32)
        end_col = jnp.sum(eye * endf, axis=1, keepdims=True)
        end_blk = (end_col / float(TM)).astype(jnp.int32)
        jblk = jax.lax.broadcasted_iota(jnp.int32, (E, NB), 1)
        bex = jnp.sum((end_blk <= jblk).astype(jnp.int32), axis=0,
                      keepdims=True)
        bex_ref[...] = jnp.minimum(bex, E - 1)


def _router(xf, Wr, br2):
    return pl.pallas_call(
        _router_body,
        grid=(NBR,),
        in_specs=[
            pl.BlockSpec((TB, D), lambda b: (b, 0)),
            pl.BlockSpec((D, E), lambda b: (0, 0)),
            pl.BlockSpec((1, E), lambda b: (0, 0)),
        ],
        out_specs=[
            pl.BlockSpec((TB, E), lambda b: (b, 0)),
            pl.BlockSpec((1, 1, TB), lambda b: (b, 0, 0)),
            pl.BlockSpec((1, 1, TB), lambda b: (b, 0, 0)),
            pl.BlockSpec((1, E), lambda b: (0, 0)),
            pl.BlockSpec((1, NB), lambda b: (0, 0)),
        ],
        out_shape=[
            jax.ShapeDtypeStruct((T, E), jnp.float32),
            jax.ShapeDtypeStruct((NBR, 1, TB), jnp.int32),
            jax.ShapeDtypeStruct((NBR, 1, TB), jnp.int32),
            jax.ShapeDtypeStruct((1, E), jnp.int32),
            jax.ShapeDtypeStruct((1, NB), jnp.int32),
        ],
        scratch_shapes=[pltpu.VMEM((1, E), jnp.float32)],
    )(xf, Wr, br2)


@functools.lru_cache(maxsize=None)
def _get_dispatch():
    mesh = plsc.VectorSubcoreMesh(core_axis_name="c", subcore_axis_name="s")

    @functools.partial(
        pl.kernel,
        out_type=[
            jax.ShapeDtypeStruct((NW, NCH, CH), jnp.int32),
            jax.ShapeDtypeStruct((NSLOT, D), jnp.float32),
        ],
        mesh=mesh,
        scratch_types=[
            pltpu.VMEM((TPW,), jnp.int32),
            pltpu.VMEM((TPW,), jnp.int32),
            pltpu.VMEM((16,), jnp.int32),
            pltpu.VMEM((NCH, CH), jnp.int32),
            pltpu.VMEM((CH, D), jnp.float32),
            pltpu.SemaphoreType.DMA,
        ],
    )
    def _dispatch(sel_hbm, rank_hbm, offs_hbm, x_hbm, si_hbm, xs_hbm,
                  sel_v, rank_v, offs_v, si_v, xrow_v, sem):
        w = lax.axis_index("s") * 2 + lax.axis_index("c")
        base = w * TPW
        pltpu.sync_copy(sel_hbm.at[pl.ds(base, TPW)], sel_v)
        pltpu.sync_copy(rank_hbm.at[pl.ds(base, TPW)], rank_v)
        pltpu.sync_copy(offs_hbm, offs_v)
        ovec = offs_v[...]
        for i in range(TPW // 16):
            s16 = sel_v[pl.ds(i * 16, 16)]
            o16 = lax.gather(
                ovec, s16[:, None],
                lax.GatherDimensionNumbers(
                    offset_dims=(), collapsed_slice_dims=(0,),
                    start_index_map=(0,)),
                slice_sizes=(1,),
                mode=lax.GatherScatterMode.PROMISE_IN_BOUNDS)
            r16 = rank_v[pl.ds(i * 16, 16)]
            si_v[i // 2, pl.ds((i % 2) * 16, 16)] = o16 + r16
        pltpu.sync_copy(si_v, si_hbm.at[w])
        for c in range(NCH):
            pltpu.sync_copy(x_hbm.at[pl.ds(base + c * CH, CH)], xrow_v)
            pltpu.async_copy(xrow_v, xs_hbm.at[si_v.at[c]], sem).wait()

    return _dispatch


def _ffn_body(bex_ref, xs_ref, w1_ref, b1_ref, w2_ref, b2_ref, out_ref):
    xb = xs_ref[...].astype(jnp.bfloat16)
    h = jnp.dot(xb, w1_ref[0], preferred_element_type=jnp.float32)
    h = jnp.maximum(h + b1_ref[0], 0.0)
    part = jnp.dot(h.astype(jnp.bfloat16), w2_ref[0],
                   preferred_element_type=jnp.float32)
    out_ref[...] = part + b2_ref[0]


def _ffn(bex, xs, W1, b1r, W2, b2r):
    grid_spec = pltpu.PrefetchScalarGridSpec(
        num_scalar_prefetch=1,
        grid=(NB,),
        in_specs=[
            pl.BlockSpec((TM, D), lambda m, be: (m, 0)),
            pl.BlockSpec((1, D, F), lambda m, be: (be[m], 0, 0)),
            pl.BlockSpec((1, 1, F), lambda m, be: (be[m], 0, 0)),
            pl.BlockSpec((1, F, D), lambda m, be: (be[m], 0, 0)),
            pl.BlockSpec((1, 1, D), lambda m, be: (be[m], 0, 0)),
        ],
        out_specs=pl.BlockSpec((TM, D), lambda m, be: (m, 0)),
    )
    return pl.pallas_call(
        _ffn_body,
        grid_spec=grid_spec,
        out_shape=jax.ShapeDtypeStruct((NSLOT, D), jnp.float32),
    )(bex, xs, W1, b1r, W2, b2r)


@functools.lru_cache(maxsize=None)
def _get_combine():
    mesh = plsc.VectorSubcoreMesh(core_axis_name="c", subcore_axis_name="s")

    @functools.partial(
        pl.kernel,
        out_type=jax.ShapeDtypeStruct((T, D), jnp.float32),
        mesh=mesh,
        scratch_types=[
            pltpu.VMEM((NCH, CH), jnp.int32),
            pltpu.VMEM((CH, D), jnp.float32),
            pltpu.SemaphoreType.DMA,
        ],
    )
    def _combine(si_hbm, ys_hbm, out_hbm, si_v, rows_v, sem):
        w = lax.axis_index("s") * 2 + lax.axis_index("c")
        pltpu.sync_copy(si_hbm.at[w], si_v)
        for c in range(NCH):
            pltpu.async_copy(ys_hbm.at[si_v.at[c]], rows_v, sem).wait()
            pltpu.sync_copy(rows_v, out_hbm.at[pl.ds(w * TPW + c * CH, CH)])

    return _combine


@jax.jit
def kernel(x, Wr, br, W1, b1, W2, b2):
    bs, ts, dim = x.shape
    xf = x.reshape(T, D)
    probs, sel3, rank3, offs, bex = _router(xf, Wr, br.reshape(1, E))
    sel = sel3.reshape(T)
    rank = rank3.reshape(T)
    offs16 = jnp.concatenate([offs.reshape(E), jnp.zeros((8,), jnp.int32)])
    si, xs = _get_dispatch()(sel, rank, offs16, xf)
    ys = _ffn(bex.reshape(NB), xs, W1.astype(jnp.bfloat16),
              b1.reshape(E, 1, F), W2.astype(jnp.bfloat16),
              b2.reshape(E, 1, D))
    final = _get_combine()(si, ys)
    return (final.reshape(bs, ts, dim),
            probs.reshape(bs, ts, E),
            sel.reshape(bs, ts))


# speedup vs baseline: 1.5594x; 1.5594x over previous
import functools

import jax
import jax.numpy as jnp
from jax import lax
from jax.experimental import pallas as pl
from jax.experimental.pallas import tpu as pltpu
from jax.experimental.pallas import tpu_sc as plsc

T = 8192
D = 1024
E = 8
F = 4096

TB = 1024
NBR = T // TB

TM = 256
NSLOT = T + E * TM
NB = NSLOT // TM
KBLK = 512
KC = F // KBLK

NW = 32
TPW = T // NW
CH = 32
NCH = TPW // CH


def _router_body(x_ref, wr_ref, br_ref,
                 probs_ref, sel_ref, rank_ref, offs_ref, bex_ref,
                 carry_ref):
    b = pl.program_id(0)
    x = x_ref[...]
    logits = jnp.dot(x, wr_ref[...], preferred_element_type=jnp.float32)
    logits = logits + br_ref[...]
    m = jnp.max(logits, axis=1, keepdims=True)
    ex = jnp.exp(logits - m)
    probs = ex / jnp.sum(ex, axis=1, keepdims=True)
    probs_ref[...] = probs
    sel = jnp.argmax(probs, axis=1).astype(jnp.int32)
    sel_ref[0, 0, :] = sel

    onehot = (jax.lax.broadcasted_iota(jnp.int32, (TB, E), 1)
              == sel[:, None]).astype(jnp.float32)
    tri = (jax.lax.broadcasted_iota(jnp.int32, (TB, TB), 1)
           <= jax.lax.broadcasted_iota(jnp.int32, (TB, TB), 0)
           ).astype(jnp.float32)
    incl = jnp.dot(tri, onehot, preferred_element_type=jnp.float32)
    rank_in = jnp.sum(incl * onehot, axis=1) - 1.0

    @pl.when(b == 0)
    def _():
        carry_ref[...] = jnp.zeros_like(carry_ref)

    carry = carry_ref[...]
    prev = jnp.sum(onehot * carry, axis=1)
    rank_ref[0, 0, :] = (rank_in + prev).astype(jnp.int32)
    carry_ref[...] = carry + jnp.sum(onehot, axis=0, keepdims=True)

    @pl.when(b == NBR - 1)
    def _():
        cnt = carry_ref[...].astype(jnp.int32)
        pc = ((cnt + (TM - 1)) // TM) * TM
        upper = (jax.lax.broadcasted_iota(jnp.int32, (E, E), 0)
                 < jax.lax.broadcasted_iota(jnp.int32, (E, E), 1)
                 ).astype(jnp.float32)
        offs = jnp.dot(pc.astype(jnp.float32), upper,
                       preferred_element_type=jnp.float32)
        offs_ref[...] = offs.astype(jnp.int32)
        endf = offs + pc.astype(jnp.float32)
        eye = (jax.lax.broadcasted_iota(jnp.int32, (E, E), 0)
               == jax.lax.broadcasted_iota(jnp.int32, (E, E), 1)
               ).astype(jnp.float32)
        end_col = jnp.sum(eye * endf, axis=1, keepdims=True)
        end_blk = (end_col / float(TM)).astype(jnp.int32)
        jblk = jax.lax.broadcasted_iota(jnp.int32, (E, NB), 1)
        bex = jnp.sum((end_blk <= jblk).astype(jnp.int32), axis=0,
                      keepdims=True)
        bex_ref[...] = jnp.minimum(bex, E - 1)


def _router(xf, Wr, br2):
    return pl.pallas_call(
        _router_body,
        grid=(NBR,),
        in_specs=[
            pl.BlockSpec((TB, D), lambda b: (b, 0)),
            pl.BlockSpec((D, E), lambda b: (0, 0)),
            pl.BlockSpec((1, E), lambda b: (0, 0)),
        ],
        out_specs=[
            pl.BlockSpec((TB, E), lambda b: (b, 0)),
            pl.BlockSpec((1, 1, TB), lambda b: (b, 0, 0)),
            pl.BlockSpec((1, 1, TB), lambda b: (b, 0, 0)),
            pl.BlockSpec((1, E), lambda b: (0, 0)),
            pl.BlockSpec((1, NB), lambda b: (0, 0)),
        ],
        out_shape=[
            jax.ShapeDtypeStruct((T, E), jnp.float32),
            jax.ShapeDtypeStruct((NBR, 1, TB), jnp.int32),
            jax.ShapeDtypeStruct((NBR, 1, TB), jnp.int32),
            jax.ShapeDtypeStruct((1, E), jnp.int32),
            jax.ShapeDtypeStruct((1, NB), jnp.int32),
        ],
        scratch_shapes=[pltpu.VMEM((1, E), jnp.float32)],
    )(xf, Wr, br2)


@functools.lru_cache(maxsize=None)
def _get_dispatch():
    mesh = plsc.VectorSubcoreMesh(core_axis_name="c", subcore_axis_name="s")

    @functools.partial(
        pl.kernel,
        out_type=[
            jax.ShapeDtypeStruct((NW, NCH, CH), jnp.int32),
            jax.ShapeDtypeStruct((NSLOT, D), jnp.float32),
        ],
        mesh=mesh,
        scratch_types=[
            pltpu.VMEM((TPW,), jnp.int32),
            pltpu.VMEM((TPW,), jnp.int32),
            pltpu.VMEM((16,), jnp.int32),
            pltpu.VMEM((NCH, CH), jnp.int32),
            pltpu.VMEM((CH, D), jnp.float32),
            pltpu.SemaphoreType.DMA,
        ],
    )
    def _dispatch(sel_hbm, rank_hbm, offs_hbm, x_hbm, si_hbm, xs_hbm,
                  sel_v, rank_v, offs_v, si_v, xrow_v, sem):
        w = lax.axis_index("s") * 2 + lax.axis_index("c")
        base = w * TPW
        pltpu.sync_copy(sel_hbm.at[pl.ds(base, TPW)], sel_v)
        pltpu.sync_copy(rank_hbm.at[pl.ds(base, TPW)], rank_v)
        pltpu.sync_copy(offs_hbm, offs_v)
        ovec = offs_v[...]
        for i in range(TPW // 16):
            s16 = sel_v[pl.ds(i * 16, 16)]
            o16 = lax.gather(
                ovec, s16[:, None],
                lax.GatherDimensionNumbers(
                    offset_dims=(), collapsed_slice_dims=(0,),
                    start_index_map=(0,)),
                slice_sizes=(1,),
                mode=lax.GatherScatterMode.PROMISE_IN_BOUNDS)
            r16 = rank_v[pl.ds(i * 16, 16)]
            si_v[i // 2, pl.ds((i % 2) * 16, 16)] = o16 + r16
        pltpu.sync_copy(si_v, si_hbm.at[w])
        for c in range(NCH):
            pltpu.sync_copy(x_hbm.at[pl.ds(base + c * CH, CH)], xrow_v)
            pltpu.async_copy(xrow_v, xs_hbm.at[si_v.at[c]], sem).wait()

    return _dispatch


def _ffn_body(bex_ref, xs_ref, w1_ref, b1_ref, w2_ref, b2_ref, out_ref):
    xb = xs_ref[...].astype(jnp.bfloat16)
    h = jnp.dot(xb, w1_ref[0], preferred_element_type=jnp.float32)
    h = jnp.maximum(h + b1_ref[0], 0.0)
    part = jnp.dot(h.astype(jnp.bfloat16), w2_ref[0],
                   preferred_element_type=jnp.float32)
    out_ref[...] = part + b2_ref[0]


def _ffn(bex, xs, W1, b1r, W2, b2r):
    grid_spec = pltpu.PrefetchScalarGridSpec(
        num_scalar_prefetch=1,
        grid=(NB,),
        in_specs=[
            pl.BlockSpec((TM, D), lambda m, be: (m, 0)),
            pl.BlockSpec((1, D, F), lambda m, be: (be[m], 0, 0)),
            pl.BlockSpec((1, 1, F), lambda m, be: (be[m], 0, 0)),
            pl.BlockSpec((1, F, D), lambda m, be: (be[m], 0, 0)),
            pl.BlockSpec((1, 1, D), lambda m, be: (be[m], 0, 0)),
        ],
        out_specs=pl.BlockSpec((TM, D), lambda m, be: (m, 0)),
    )
    return pl.pallas_call(
        _ffn_body,
        grid_spec=grid_spec,
        out_shape=jax.ShapeDtypeStruct((NSLOT, D), jnp.float32),
    )(bex, xs, W1, b1r, W2, b2r)


@functools.lru_cache(maxsize=None)
def _get_combine():
    mesh = plsc.VectorSubcoreMesh(core_axis_name="c", subcore_axis_name="s")

    @functools.partial(
        pl.kernel,
        out_type=jax.ShapeDtypeStruct((T, D), jnp.float32),
        mesh=mesh,
        scratch_types=[
            pltpu.VMEM((NCH, CH), jnp.int32),
            pltpu.VMEM((CH, D), jnp.float32),
            pltpu.SemaphoreType.DMA,
        ],
    )
    def _combine(si_hbm, ys_hbm, out_hbm, si_v, rows_v, sem):
        w = lax.axis_index("s") * 2 + lax.axis_index("c")
        pltpu.sync_copy(si_hbm.at[w], si_v)
        for c in range(NCH):
            pltpu.async_copy(ys_hbm.at[si_v.at[c]], rows_v, sem).wait()
            pltpu.sync_copy(rows_v, out_hbm.at[pl.ds(w * TPW + c * CH, CH)])

    return _combine


@jax.jit
def kernel(x, Wr, br, W1, b1, W2, b2):
    bs, ts, dim = x.shape
    xf = x.reshape(T, D)
    probs, sel3, rank3, offs, bex = _router(xf, Wr, br.reshape(1, E))
    sel = sel3.reshape(T)
    rank = rank3.reshape(T)
    offs16 = jnp.concatenate([offs.reshape(E), jnp.zeros((8,), jnp.int32)])
    si, xs = _get_dispatch()(sel, rank, offs16, xf)
    ys = _ffn(bex.reshape(NB), xs, W1.astype(jnp.bfloat16),
              b1.reshape(E, 1, F), W2.astype(jnp.bfloat16),
              b2.reshape(E, 1, D))
    final = _get_combine()(si, ys)
    return (final.reshape(bs, ts, dim),
            probs.reshape(bs, ts, E),
            sel.reshape(bs, ts))


# speedup vs baseline: 1.5673x; 1.0051x over previous
import functools

import jax
import jax.numpy as jnp
from jax import lax
from jax.experimental import pallas as pl
from jax.experimental.pallas import tpu as pltpu
from jax.experimental.pallas import tpu_sc as plsc

T = 8192
D = 1024
E = 8
F = 4096

TB = 1024
NBR = T // TB

TM = 256
NSLOT = T + E * TM
NB = NSLOT // TM
KBLK = 512
KC = F // KBLK

NW = 32
TPW = T // NW
CH = 32
NCH = TPW // CH


def _router_body(x_ref, wr_ref, br_ref,
                 probs_ref, sel_ref, rank_ref, offs_ref, bex_ref,
                 carry_ref):
    b = pl.program_id(0)
    x = x_ref[...]
    logits = jnp.dot(x, wr_ref[...], preferred_element_type=jnp.float32)
    logits = logits + br_ref[...]
    m = jnp.max(logits, axis=1, keepdims=True)
    ex = jnp.exp(logits - m)
    probs = ex / jnp.sum(ex, axis=1, keepdims=True)
    probs_ref[...] = probs
    sel = jnp.argmax(probs, axis=1).astype(jnp.int32)
    sel_ref[...] = sel.reshape(TB // 128, 128)

    onehot = (jax.lax.broadcasted_iota(jnp.int32, (TB, E), 1)
              == sel[:, None]).astype(jnp.float32)
    tri = (jax.lax.broadcasted_iota(jnp.int32, (TB, TB), 1)
           <= jax.lax.broadcasted_iota(jnp.int32, (TB, TB), 0)
           ).astype(jnp.float32)
    incl = jnp.dot(tri, onehot, preferred_element_type=jnp.float32)
    rank_in = jnp.sum(incl * onehot, axis=1) - 1.0

    @pl.when(b == 0)
    def _():
        carry_ref[...] = jnp.zeros_like(carry_ref)

    carry = carry_ref[...]
    prev = jnp.sum(onehot * carry, axis=1)
    rank_ref[...] = (rank_in + prev).astype(jnp.int32).reshape(TB // 128, 128)
    carry_ref[...] = carry + jnp.sum(onehot, axis=0, keepdims=True)

    @pl.when(b == NBR - 1)
    def _():
        cnt = carry_ref[...].astype(jnp.int32)
        pc = ((cnt + (TM - 1)) // TM) * TM
        upper = (jax.lax.broadcasted_iota(jnp.int32, (E, E), 0)
                 < jax.lax.broadcasted_iota(jnp.int32, (E, E), 1)
                 ).astype(jnp.float32)
        offs = jnp.dot(pc.astype(jnp.float32), upper,
                       preferred_element_type=jnp.float32)
        offs_ref[...] = offs.astype(jnp.int32)
        endf = offs + pc.astype(jnp.float32)
        eye = (jax.lax.broadcasted_iota(jnp.int32, (E, E), 0)
               == jax.lax.broadcasted_iota(jnp.int32, (E, E), 1)
               ).astype(jnp.float32)
        end_col = jnp.sum(eye * endf, axis=1, keepdims=True)
        end_blk = (end_col / float(TM)).astype(jnp.int32)
        jblk = jax.lax.broadcasted_iota(jnp.int32, (E, NB), 1)
        bex = jnp.sum((end_blk <= jblk).astype(jnp.int32), axis=0,
                      keepdims=True)
        bex_ref[...] = jnp.minimum(bex, E - 1)


def _router(xf, Wr, br2):
    return pl.pallas_call(
        _router_body,
        grid=(NBR,),
        in_specs=[
            pl.BlockSpec((TB, D), lambda b: (b, 0)),
            pl.BlockSpec((D, E), lambda b: (0, 0)),
            pl.BlockSpec((1, E), lambda b: (0, 0)),
        ],
        out_specs=[
            pl.BlockSpec((TB, E), lambda b: (b, 0)),
            pl.BlockSpec((TB // 128, 128), lambda b: (b, 0)),
            pl.BlockSpec((TB // 128, 128), lambda b: (b, 0)),
            pl.BlockSpec((1, E), lambda b: (0, 0)),
            pl.BlockSpec((1, NB), lambda b: (0, 0)),
        ],
        out_shape=[
            jax.ShapeDtypeStruct((T, E), jnp.float32),
            jax.ShapeDtypeStruct((T // 128, 128), jnp.int32),
            jax.ShapeDtypeStruct((T // 128, 128), jnp.int32),
            jax.ShapeDtypeStruct((1, E), jnp.int32),
            jax.ShapeDtypeStruct((1, NB), jnp.int32),
        ],
        scratch_shapes=[pltpu.VMEM((1, E), jnp.float32)],
    )(xf, Wr, br2)


@functools.lru_cache(maxsize=None)
def _get_dispatch():
    mesh = plsc.VectorSubcoreMesh(core_axis_name="c", subcore_axis_name="s")

    @functools.partial(
        pl.kernel,
        out_type=[
            jax.ShapeDtypeStruct((NW, NCH, CH), jnp.int32),
            jax.ShapeDtypeStruct((NSLOT, D), jnp.float32),
        ],
        mesh=mesh,
        scratch_types=[
            pltpu.VMEM((TPW,), jnp.int32),
            pltpu.VMEM((TPW,), jnp.int32),
            pltpu.VMEM((16,), jnp.int32),
            pltpu.VMEM((NCH, CH), jnp.int32),
            pltpu.VMEM((CH, D), jnp.float32),
            pltpu.SemaphoreType.DMA,
        ],
    )
    def _dispatch(sel_hbm, rank_hbm, offs_hbm, x_hbm, si_hbm, xs_hbm,
                  sel_v, rank_v, offs_v, si_v, xrow_v, sem):
        w = lax.axis_index("s") * 2 + lax.axis_index("c")
        base = w * TPW
        pltpu.sync_copy(sel_hbm.at[pl.ds(base, TPW)], sel_v)
        pltpu.sync_copy(rank_hbm.at[pl.ds(base, TPW)], rank_v)
        pltpu.sync_copy(offs_hbm, offs_v)
        ovec = offs_v[...]
        for i in range(TPW // 16):
            s16 = sel_v[pl.ds(i * 16, 16)]
            o16 = lax.gather(
                ovec, s16[:, None],
                lax.GatherDimensionNumbers(
                    offset_dims=(), collapsed_slice_dims=(0,),
                    start_index_map=(0,)),
                slice_sizes=(1,),
                mode=lax.GatherScatterMode.PROMISE_IN_BOUNDS)
            r16 = rank_v[pl.ds(i * 16, 16)]
            si_v[i // 2, pl.ds((i % 2) * 16, 16)] = o16 + r16
        pltpu.sync_copy(si_v, si_hbm.at[w])
        for c in range(NCH):
            pltpu.sync_copy(x_hbm.at[pl.ds(base + c * CH, CH)], xrow_v)
            pltpu.async_copy(xrow_v, xs_hbm.at[si_v.at[c]], sem).wait()

    return _dispatch


def _ffn_body(bex_ref, xs_ref, w1_ref, b1_ref, w2_ref, b2_ref, out_ref):
    xb = xs_ref[...].astype(jnp.bfloat16)
    h = jnp.dot(xb, w1_ref[0], preferred_element_type=jnp.float32)
    h = jnp.maximum(h + b1_ref[0], 0.0)
    part = jnp.dot(h.astype(jnp.bfloat16), w2_ref[0],
                   preferred_element_type=jnp.float32)
    out_ref[...] = part + b2_ref[0]


def _ffn(bex, xs, W1, b1r, W2, b2r):
    grid_spec = pltpu.PrefetchScalarGridSpec(
        num_scalar_prefetch=1,
        grid=(NB,),
        in_specs=[
            pl.BlockSpec((TM, D), lambda m, be: (m, 0)),
            pl.BlockSpec((1, D, F), lambda m, be: (be[m], 0, 0)),
            pl.BlockSpec((1, 1, F), lambda m, be: (be[m], 0, 0)),
            pl.BlockSpec((1, F, D), lambda m, be: (be[m], 0, 0)),
            pl.BlockSpec((1, 1, D), lambda m, be: (be[m], 0, 0)),
        ],
        out_specs=pl.BlockSpec((TM, D), lambda m, be: (m, 0)),
    )
    return pl.pallas_call(
        _ffn_body,
        grid_spec=grid_spec,
        out_shape=jax.ShapeDtypeStruct((NSLOT, D), jnp.float32),
    )(bex, xs, W1, b1r, W2, b2r)


@functools.lru_cache(maxsize=None)
def _get_combine():
    mesh = plsc.VectorSubcoreMesh(core_axis_name="c", subcore_axis_name="s")

    @functools.partial(
        pl.kernel,
        out_type=jax.ShapeDtypeStruct((T, D), jnp.float32),
        mesh=mesh,
        scratch_types=[
            pltpu.VMEM((NCH, CH), jnp.int32),
            pltpu.VMEM((CH, D), jnp.float32),
            pltpu.SemaphoreType.DMA,
        ],
    )
    def _combine(si_hbm, ys_hbm, out_hbm, si_v, rows_v, sem):
        w = lax.axis_index("s") * 2 + lax.axis_index("c")
        pltpu.sync_copy(si_hbm.at[w], si_v)
        for c in range(NCH):
            pltpu.async_copy(ys_hbm.at[si_v.at[c]], rows_v, sem).wait()
            pltpu.sync_copy(rows_v, out_hbm.at[pl.ds(w * TPW + c * CH, CH)])

    return _combine


@jax.jit
def kernel(x, Wr, br, W1, b1, W2, b2):
    bs, ts, dim = x.shape
    xf = x.reshape(T, D)
    probs, sel3, rank3, offs, bex = _router(xf, Wr, br.reshape(1, E))
    sel = sel3.reshape(T)
    rank = rank3.reshape(T)
    offs16 = jnp.concatenate([offs.reshape(E), jnp.zeros((8,), jnp.int32)])
    si, xs = _get_dispatch()(sel, rank, offs16, xf)
    ys = _ffn(bex.reshape(NB), xs, W1.astype(jnp.bfloat16),
              b1.reshape(E, 1, F), W2.astype(jnp.bfloat16),
              b2.reshape(E, 1, D))
    final = _get_combine()(si, ys)
    return (final.reshape(bs, ts, dim),
            probs.reshape(bs, ts, E),
            sel.reshape(bs, ts))
